# baseline (speedup 1.0000x reference)
import functools
import math

import jax
import jax.numpy as jnp
from jax import lax
from jax.experimental import pallas as pl
from jax.experimental.pallas import tpu as pltpu
from jax.experimental.pallas import tpu_sc as plsc

TOPK = 8
B = 8
T = 60
N = 196
D = 512
H = 4
HD = D // H

NC = 2
NS = 16
NW = NC * NS
ROWS = B * TOPK * N
RPW = ROWS // NW
CHUNK = 56
NCHUNK = RPW // CHUNK


def _topk_idx_kernel(q_ref, key_ref, wq_ref, wk_ref, bq_ref, bk_ref, out_ref):
    b = pl.program_id(0)
    q = q_ref[0]
    kfeat = key_ref[0]
    dn = (((1,), (1,)), ((), ()))
    Q = lax.dot_general(kfeat, wq_ref[...], dn,
                        preferred_element_type=jnp.float32,
                        precision=lax.Precision.HIGHEST) + bq_ref[...]
    K = lax.dot_general(q, wk_ref[...], dn,
                        preferred_element_type=jnp.float32,
                        precision=lax.Precision.HIGHEST) + bk_ref[...]
    KQ = K * Q
    scale = 1.0 / math.sqrt(HD)
    iota_t = lax.broadcasted_iota(jnp.int32, (T, 1), 0)
    tw = jnp.zeros((T, 1), jnp.float32)
    for h in range(H):
        s = jnp.sum(KQ[:, h * HD:(h + 1) * HD], axis=1, keepdims=True) * scale
        m = jnp.max(s, axis=0, keepdims=True)
        e = jnp.exp(s - m)
        tw = tw + e / jnp.sum(e, axis=0, keepdims=True)

    sel = iota_t < 0
    cur = tw
    for _ in range(TOPK):
        vmax = jnp.max(cur, axis=0, keepdims=True)
        cand = jnp.where(cur >= vmax, iota_t, -1)
        pick = jnp.max(cand, axis=0, keepdims=True)
        picked = iota_t == pick
        sel = sel | picked
        cur = jnp.where(picked, -jnp.inf, cur)

    lane = lax.broadcasted_iota(jnp.int32, (1, TOPK * N), 1)
    acc = jnp.zeros((1, TOPK * N), jnp.int32)
    mask = sel
    for k in range(TOPK):
        t_k = jnp.min(jnp.where(mask, iota_t, T + 1), axis=0, keepdims=True)
        mask = mask & (iota_t != t_k)
        in_rng = (lane >= k * N) & (lane < (k + 1) * N)
        base = (b * T + t_k) * N - k * N
        acc = acc + jnp.where(in_rng, base, 0)
    out_ref[0] = acc + lane


def _compute_gather_indices(query, key, wq, wk, bq, bk):
    out = pl.pallas_call(
        _topk_idx_kernel,
        grid=(B,),
        in_specs=[
            pl.BlockSpec((1, T, D), lambda b: (b, 0, 0)),
            pl.BlockSpec((1, 1, D), lambda b: (b, 0, 0)),
            pl.BlockSpec((D, D), lambda b: (0, 0)),
            pl.BlockSpec((D, D), lambda b: (0, 0)),
            pl.BlockSpec((1, D), lambda b: (0, 0)),
            pl.BlockSpec((1, D), lambda b: (0, 0)),
        ],
        out_specs=pl.BlockSpec((1, 1, TOPK * N), lambda b: (b, 0, 0)),
        out_shape=jax.ShapeDtypeStruct((B, 1, TOPK * N), jnp.int32),
    )(query, key.reshape(B, 1, D), wq, wk, bq, bk)
    return out.reshape(ROWS)


def _sc_gather_body(value_hbm, idx_hbm, out_hbm, idx_v, row_v, sem):
    wid = lax.axis_index("s") * NC + lax.axis_index("c")
    base = wid * RPW
    for c in range(NCHUNK):
        off = base + c * CHUNK
        pltpu.sync_copy(idx_hbm.at[pl.ds(off, CHUNK)], idx_v)
        pltpu.async_copy(value_hbm.at[idx_v], row_v, sem).wait()
        pltpu.sync_copy(row_v, out_hbm.at[pl.ds(off, CHUNK)])


@functools.cache
def _make_sc_gather():
    return pl.kernel(
        _sc_gather_body,
        out_type=jax.ShapeDtypeStruct((ROWS, D), jnp.float32),
        mesh=plsc.VectorSubcoreMesh(
            core_axis_name="c", subcore_axis_name="s",
            num_cores=NC, num_subcores=NS),
        scratch_types=[
            pltpu.VMEM((CHUNK,), jnp.int32),
            pltpu.VMEM((CHUNK, D), jnp.float32),
            pltpu.SemaphoreType.DMA,
        ],
    )


def kernel(query, key, value, in_proj_w, in_proj_b, out_proj_w, out_proj_b,
           lin1_w, lin1_b, lin2_w, lin2_b, ln_w, ln_b):
    wq = in_proj_w[:D]
    wk = in_proj_w[D:2 * D]
    bq = in_proj_b[:D].reshape(1, D)
    bk = in_proj_b[D:2 * D].reshape(1, D)
    idx_flat = _compute_gather_indices(query, key, wq, wk, bq, bk)
    value2d = value.reshape(B * T * N, D)
    out2d = _make_sc_gather()(value2d, idx_flat)
    return out2d.reshape(B, TOPK, N, D)
